# baseline (speedup 1.0000x reference)
import functools

import jax
import jax.numpy as jnp
from jax import lax
from jax.experimental import pallas as pl
from jax.experimental.pallas import tpu as pltpu

try:
    from jax.experimental.pallas import tpu_sc as plsc
    _HAS_SC = True
except ImportError:
    _HAS_SC = False

ROW_TILE = 256


def _argmin_body(x_ref, e_ref, idx_ref, acc_ref, en_ref, sc_ref, xn_ref):
    i = pl.program_id(0)
    nt = pl.num_programs(0) - 1
    k = e_ref.shape[0]
    rt = x_ref.shape[0]
    rh = rt // 2
    lw = 128
    ncol = k // lw
    par = lax.rem(i, 2)
    opar = 1 - par

    @pl.when(i == 0)
    def _():
        e = e_ref[...]
        en_ref[...] = jnp.sum(e * e, axis=1)[None, :]
        acc_ref[...] = jnp.zeros_like(acc_ref)

    x = x_ref[...]
    sc_ref[par] = lax.dot_general(
        2.0 * x, e_ref[...], (((1,), (1,)), ((), ())),
        preferred_element_type=jnp.float32)
    xn_ref[par] = jnp.sum(x * x, axis=1, keepdims=True)

    en = en_ref[...]
    lane = lax.broadcasted_iota(jnp.int32, (rh, lw), 1)
    loss = jnp.zeros((), jnp.float32)
    for h in range(2):
        r0 = h * rh
        xn_h = xn_ref[opar, r0:r0 + rh, :]
        r_val = jnp.full((rh, lw), jnp.inf, jnp.float32)
        r_col = jnp.zeros((rh, lw), jnp.int32)
        for col in range(ncol):
            s = sc_ref[opar, r0:r0 + rh, col * lw:(col + 1) * lw]
            cc = xn_h - s + en[:, col * lw:(col + 1) * lw]
            lt = cc < r_val
            r_val = jnp.where(lt, cc, r_val)
            r_col = jnp.where(lt, col, r_col)
        mn = jnp.min(r_val, axis=1, keepdims=True)
        kc = r_col * lw + lane
        idx = jnp.min(jnp.where(r_val <= mn, kc, k), axis=1)
        idx_ref[0, 0, r0:r0 + rh] = idx
        loss = loss + jnp.sum(mn)
    acc_ref[...] += jnp.where(i > 0, loss, 0.0)


def _argmin_call(flat, embed):
    m, d = flat.shape
    k = embed.shape[0]
    nt = m // ROW_TILE
    grid = (nt + 1,)
    idx_out = jax.ShapeDtypeStruct((nt, 1, ROW_TILE), jnp.int32)
    acc_out = jax.ShapeDtypeStruct((1, 1), jnp.float32)
    return pl.pallas_call(
        _argmin_body,
        grid=grid,
        in_specs=[
            pl.BlockSpec((ROW_TILE, d), lambda i: (jnp.minimum(i, nt - 1), 0)),
            pl.BlockSpec((k, d), lambda i: (0, 0)),
        ],
        out_specs=[
            pl.BlockSpec((1, 1, ROW_TILE), lambda i: ((i + nt - 1) % nt, 0, 0)),
            pl.BlockSpec((1, 1), lambda i: (0, 0)),
        ],
        out_shape=[idx_out, acc_out],
        scratch_shapes=[
            pltpu.VMEM((1, k), jnp.float32),
            pltpu.VMEM((2, ROW_TILE, k), jnp.float32),
            pltpu.VMEM((2, ROW_TILE, 1), jnp.float32),
        ],
    )(flat, embed)


def _make_gather(k, d, b):
    info = plsc.get_sparse_core_info()
    nw = info.num_cores * info.num_subcores
    ch = 128
    b_per_w = b // nw
    n_chunks = b_per_w // ch
    mesh = plsc.VectorSubcoreMesh(core_axis_name="c", subcore_axis_name="s")

    @functools.partial(
        pl.kernel,
        mesh=mesh,
        out_type=jax.ShapeDtypeStruct((b, d), jnp.float32),
        scratch_types=[
            pltpu.VMEM((ch,), jnp.int32),
            pltpu.VMEM((ch, d), jnp.float32),
            pltpu.SemaphoreType.DMA,
        ],
    )
    def gather_k(table_hbm, idx_hbm, out_hbm, idx_v, rows_v, sem):
        wid = lax.axis_index("s") * info.num_cores + lax.axis_index("c")
        base = wid * b_per_w

        def chunk(c, carry):
            off = base + c * ch
            pltpu.sync_copy(idx_hbm.at[pl.ds(off, ch)], idx_v)
            pltpu.async_copy(table_hbm.at[idx_v], rows_v, sem).wait()
            pltpu.sync_copy(rows_v, out_hbm.at[pl.ds(off, ch)])
            return carry

        lax.fori_loop(0, n_chunks, chunk, 0)

    return gather_k


def kernel(z, embed):
    b, n, d = z.shape
    k = embed.shape[0]
    m = b * n
    flat = z.reshape(m, d)
    idx3, acc = _argmin_call(flat, embed)
    indices = idx3.reshape(m)
    zq_flat = _make_gather(k, d, m)(embed, indices)
    loss = acc[0, 0] * (2.0 / (m * d))
    return zq_flat.reshape(b, n, d), indices.reshape(b, n), loss


# speedup vs baseline: 1.4980x; 1.4980x over previous
import functools

import jax
import jax.numpy as jnp
from jax import lax
from jax.experimental import pallas as pl
from jax.experimental.pallas import tpu as pltpu

try:
    from jax.experimental.pallas import tpu_sc as plsc
    _HAS_SC = True
except ImportError:
    _HAS_SC = False

ROW_TILE = 256


def _argmin_body(x_ref, e_ref, idx_ref, acc_ref, en_ref):
    i = pl.program_id(0)
    x = x_ref[...]
    k = e_ref.shape[0]

    @pl.when(i == 0)
    def _():
        e = e_ref[...]
        en_ref[...] = jnp.sum(e * e, axis=1)[None, :]
        acc_ref[...] = jnp.zeros_like(acc_ref)

    x2 = 2.0 * x
    xn = jnp.sum(x * x, axis=1, keepdims=True)
    en = en_ref[...]
    rt = x.shape[0]
    rh = rt // 2
    lw = 128
    ncol = k // lw
    nchunk = 8
    cpc = ncol // nchunk
    xn_h = [lax.slice(xn, (0, 0), (rh, 1)),
            lax.slice(xn, (rh, 0), (rt, 1))]
    scs = []
    r_val = [jnp.full((rh, lw), jnp.inf, jnp.float32) for _ in range(2)]
    r_col = [jnp.zeros((rh, lw), jnp.int32) for _ in range(2)]

    def scan_cols(h, c, sc_c):
        r0, r1 = h * rh, (h + 1) * rh
        for j in range(cpc):
            col = c * cpc + j
            s = lax.slice(sc_c, (r0, j * lw), (r1, (j + 1) * lw))
            cc = xn_h[h] - s + lax.slice(en, (0, col * lw), (1, (col + 1) * lw))
            lt = cc < r_val[h]
            r_val[h] = jnp.where(lt, cc, r_val[h])
            r_col[h] = jnp.where(lt, col, r_col[h])

    for c in range(nchunk):
        e_c = e_ref[pl.ds(c * cpc * lw, cpc * lw), :]
        sc_c = lax.dot_general(
            x2, e_c, (((1,), (1,)), ((), ())),
            preferred_element_type=jnp.float32)
        scs.append(sc_c)
        scan_cols(0, c, sc_c)
    for c in range(nchunk):
        scan_cols(1, c, scs[c])

    lane = lax.broadcasted_iota(jnp.int32, (rh, lw), 1)
    loss_sum = jnp.zeros((), jnp.float32)
    for h in range(2):
        mn = jnp.min(r_val[h], axis=1, keepdims=True)
        kc = r_col[h] * lw + lane
        idx = jnp.min(jnp.where(r_val[h] <= mn, kc, k), axis=1)
        idx_ref[0, 0, h * rh:(h + 1) * rh] = idx
        loss_sum = loss_sum + jnp.sum(mn)
    acc_ref[...] += loss_sum


def _argmin_call(flat, embed):
    m, d = flat.shape
    k = embed.shape[0]
    nt = m // ROW_TILE
    grid = (nt,)
    idx_out = jax.ShapeDtypeStruct((nt, 1, ROW_TILE), jnp.int32)
    acc_out = jax.ShapeDtypeStruct((1, 1), jnp.float32)
    return pl.pallas_call(
        _argmin_body,
        grid=grid,
        in_specs=[
            pl.BlockSpec((ROW_TILE, d), lambda i: (i, 0)),
            pl.BlockSpec((k, d), lambda i: (0, 0)),
        ],
        out_specs=[
            pl.BlockSpec((1, 1, ROW_TILE), lambda i: (i, 0, 0)),
            pl.BlockSpec((1, 1), lambda i: (0, 0)),
        ],
        out_shape=[idx_out, acc_out],
        scratch_shapes=[pltpu.VMEM((1, k), jnp.float32)],
    )(flat, embed)


def _make_gather(k, d, b):
    info = plsc.get_sparse_core_info()
    nw = info.num_cores * info.num_subcores
    ch = 128
    b_per_w = b // nw
    n_chunks = b_per_w // ch
    mesh = plsc.VectorSubcoreMesh(core_axis_name="c", subcore_axis_name="s")

    @functools.partial(
        pl.kernel,
        mesh=mesh,
        out_type=jax.ShapeDtypeStruct((b, d), jnp.float32),
        scratch_types=[
            pltpu.VMEM((ch,), jnp.int32),
            pltpu.VMEM((ch, d), jnp.float32),
            pltpu.SemaphoreType.DMA,
        ],
    )
    def gather_k(table_hbm, idx_hbm, out_hbm, idx_v, rows_v, sem):
        wid = lax.axis_index("s") * info.num_cores + lax.axis_index("c")
        base = wid * b_per_w

        def chunk(c, carry):
            off = base + c * ch
            pltpu.sync_copy(idx_hbm.at[pl.ds(off, ch)], idx_v)
            pltpu.async_copy(table_hbm.at[idx_v], rows_v, sem).wait()
            pltpu.sync_copy(rows_v, out_hbm.at[pl.ds(off, ch)])
            return carry

        lax.fori_loop(0, n_chunks, chunk, 0)

    return gather_k


def kernel(z, embed):
    b, n, d = z.shape
    k = embed.shape[0]
    m = b * n
    mh = m // 2
    flat = z.reshape(m, d)
    gather = _make_gather(k, d, mh)
    idx3_0, acc0 = _argmin_call(flat[:mh], embed)
    ind0 = idx3_0.reshape(mh)
    zq0 = gather(embed, ind0)
    idx3_1, acc1 = _argmin_call(flat[mh:], embed)
    ind1 = idx3_1.reshape(mh)
    zq1 = gather(embed, ind1)
    indices = jnp.concatenate([ind0, ind1])
    zq_flat = jnp.concatenate([zq0, zq1])
    loss = (acc0[0, 0] + acc1[0, 0]) * (2.0 / (m * d))
    return zq_flat.reshape(b, n, d), indices.reshape(b, n), loss


# speedup vs baseline: 1.8432x; 1.2305x over previous
import functools

import jax
import jax.numpy as jnp
from jax import lax
from jax.experimental import pallas as pl
from jax.experimental.pallas import tpu as pltpu

try:
    from jax.experimental.pallas import tpu_sc as plsc
    _HAS_SC = True
except ImportError:
    _HAS_SC = False

ROW_TILE = 256


def _argmin_body(x_ref, e_ref, idx_ref, acc_ref, en_ref, ebf_ref):
    i = pl.program_id(0)
    x = x_ref[...]
    k = e_ref.shape[0]

    @pl.when(i == 0)
    def _():
        e = e_ref[...]
        en_ref[...] = jnp.sum(e * e, axis=1)[None, :]
        ebf_ref[...] = e.astype(jnp.bfloat16)
        acc_ref[...] = jnp.zeros_like(acc_ref)

    x2 = (2.0 * x).astype(jnp.bfloat16)
    xn = jnp.sum(x * x, axis=1, keepdims=True)
    en = en_ref[...]
    rt = x.shape[0]
    rh = rt // 2
    lw = 128
    ncol = k // lw
    nchunk = 8
    cpc = ncol // nchunk
    xn_h = [lax.slice(xn, (0, 0), (rh, 1)),
            lax.slice(xn, (rh, 0), (rt, 1))]
    scs = []
    r_val = [jnp.full((rh, lw), jnp.inf, jnp.float32) for _ in range(2)]
    r_col = [jnp.zeros((rh, lw), jnp.int32) for _ in range(2)]

    def scan_cols(h, c, sc_c):
        r0, r1 = h * rh, (h + 1) * rh
        for j in range(cpc):
            col = c * cpc + j
            s = lax.slice(sc_c, (r0, j * lw), (r1, (j + 1) * lw))
            cc = xn_h[h] - s + lax.slice(en, (0, col * lw), (1, (col + 1) * lw))
            lt = cc < r_val[h]
            r_val[h] = jnp.where(lt, cc, r_val[h])
            r_col[h] = jnp.where(lt, col, r_col[h])

    for c in range(nchunk):
        e_c = ebf_ref[pl.ds(c * cpc * lw, cpc * lw), :]
        sc_c = lax.dot_general(
            x2, e_c, (((1,), (1,)), ((), ())),
            preferred_element_type=jnp.float32)
        scs.append(sc_c)
        scan_cols(0, c, sc_c)
    for c in range(nchunk):
        scan_cols(1, c, scs[c])

    lane = lax.broadcasted_iota(jnp.int32, (rh, lw), 1)
    loss_sum = jnp.zeros((), jnp.float32)
    for h in range(2):
        mn = jnp.min(r_val[h], axis=1, keepdims=True)
        kc = r_col[h] * lw + lane
        idx = jnp.min(jnp.where(r_val[h] <= mn, kc, k), axis=1)
        idx_ref[0, 0, h * rh:(h + 1) * rh] = idx
        loss_sum = loss_sum + jnp.sum(mn)
    acc_ref[...] += loss_sum


def _argmin_call(flat, embed):
    m, d = flat.shape
    k = embed.shape[0]
    nt = m // ROW_TILE
    grid = (nt,)
    idx_out = jax.ShapeDtypeStruct((nt, 1, ROW_TILE), jnp.int32)
    acc_out = jax.ShapeDtypeStruct((1, 1), jnp.float32)
    return pl.pallas_call(
        _argmin_body,
        grid=grid,
        in_specs=[
            pl.BlockSpec((ROW_TILE, d), lambda i: (i, 0)),
            pl.BlockSpec((k, d), lambda i: (0, 0)),
        ],
        out_specs=[
            pl.BlockSpec((1, 1, ROW_TILE), lambda i: (i, 0, 0)),
            pl.BlockSpec((1, 1), lambda i: (0, 0)),
        ],
        out_shape=[idx_out, acc_out],
        scratch_shapes=[pltpu.VMEM((1, k), jnp.float32),
                        pltpu.VMEM((k, d), jnp.bfloat16)],
    )(flat, embed)


def _make_gather(k, d, b):
    info = plsc.get_sparse_core_info()
    nw = info.num_cores * info.num_subcores
    ch = 128
    b_per_w = b // nw
    n_chunks = b_per_w // ch
    mesh = plsc.VectorSubcoreMesh(core_axis_name="c", subcore_axis_name="s")

    @functools.partial(
        pl.kernel,
        mesh=mesh,
        out_type=jax.ShapeDtypeStruct((b, d), jnp.float32),
        scratch_types=[
            pltpu.VMEM((ch,), jnp.int32),
            pltpu.VMEM((ch, d), jnp.float32),
            pltpu.SemaphoreType.DMA,
        ],
    )
    def gather_k(table_hbm, idx_hbm, out_hbm, idx_v, rows_v, sem):
        wid = lax.axis_index("s") * info.num_cores + lax.axis_index("c")
        base = wid * b_per_w

        def chunk(c, carry):
            off = base + c * ch
            pltpu.sync_copy(idx_hbm.at[pl.ds(off, ch)], idx_v)
            pltpu.async_copy(table_hbm.at[idx_v], rows_v, sem).wait()
            pltpu.sync_copy(rows_v, out_hbm.at[pl.ds(off, ch)])
            return carry

        lax.fori_loop(0, n_chunks, chunk, 0)

    return gather_k


def kernel(z, embed):
    b, n, d = z.shape
    k = embed.shape[0]
    m = b * n
    flat = z.reshape(m, d)
    idx3, acc = _argmin_call(flat, embed)
    indices = idx3.reshape(m)
    zq_flat = _make_gather(k, d, m)(embed, indices)
    loss = acc[0, 0] * (2.0 / (m * d))
    return zq_flat.reshape(b, n, d), indices.reshape(b, n), loss


# speedup vs baseline: 1.8445x; 1.0007x over previous
import functools

import jax
import jax.numpy as jnp
from jax import lax
from jax.experimental import pallas as pl
from jax.experimental.pallas import tpu as pltpu

try:
    from jax.experimental.pallas import tpu_sc as plsc
    _HAS_SC = True
except ImportError:
    _HAS_SC = False

ROW_TILE = 256


def _argmin_body(x_ref, e_ref, idx_ref, acc_ref, en_ref, ebf_ref):
    i = pl.program_id(0)
    x = x_ref[...]
    k = e_ref.shape[0]

    @pl.when(i == 0)
    def _():
        e = e_ref[...]
        en_ref[...] = jnp.sum(e * e, axis=1)[None, :]
        ebf_ref[...] = e.astype(jnp.bfloat16)
        acc_ref[...] = jnp.zeros_like(acc_ref)

    x2 = (2.0 * x).astype(jnp.bfloat16)
    xn = jnp.sum(x * x, axis=1, keepdims=True)
    en = en_ref[...]
    rt = x.shape[0]
    rh = rt // 2
    lw = 128
    ncol = k // lw
    nchunk = 8
    cpc = ncol // nchunk
    xn_h = [lax.slice(xn, (0, 0), (rh, 1)),
            lax.slice(xn, (rh, 0), (rt, 1))]
    scs = []
    r_val = [jnp.full((rh, lw), jnp.inf, jnp.float32) for _ in range(2)]
    r_col = [jnp.zeros((rh, lw), jnp.int32) for _ in range(2)]

    def scan_cols(h, c, sc_c):
        r0, r1 = h * rh, (h + 1) * rh
        for j in range(cpc):
            col = c * cpc + j
            s = lax.slice(sc_c, (r0, j * lw), (r1, (j + 1) * lw))
            cc = xn_h[h] - s + lax.slice(en, (0, col * lw), (1, (col + 1) * lw))
            lt = cc < r_val[h]
            r_val[h] = jnp.where(lt, cc, r_val[h])
            r_col[h] = jnp.where(lt, col, r_col[h])

    for c in range(nchunk):
        e_c = ebf_ref[pl.ds(c * cpc * lw, cpc * lw), :]
        sc_c = lax.dot_general(
            x2, e_c, (((1,), (1,)), ((), ())),
            preferred_element_type=jnp.float32)
        scs.append(sc_c)
        if c > 0:
            scan_cols(0, c - 1, scs[c - 1])
    scan_cols(0, nchunk - 1, scs[-1])
    for c in range(nchunk):
        scan_cols(1, c, scs[c])

    lane = lax.broadcasted_iota(jnp.int32, (rh, lw), 1)
    loss_sum = jnp.zeros((), jnp.float32)
    for h in range(2):
        mn = jnp.min(r_val[h], axis=1, keepdims=True)
        kc = r_col[h] * lw + lane
        idx = jnp.min(jnp.where(r_val[h] <= mn, kc, k), axis=1)
        idx_ref[0, 0, h * rh:(h + 1) * rh] = idx
        loss_sum = loss_sum + jnp.sum(mn)
    acc_ref[...] += loss_sum


def _argmin_call(flat, embed):
    m, d = flat.shape
    k = embed.shape[0]
    nt = m // ROW_TILE
    grid = (nt,)
    idx_out = jax.ShapeDtypeStruct((nt, 1, ROW_TILE), jnp.int32)
    acc_out = jax.ShapeDtypeStruct((1, 1), jnp.float32)
    return pl.pallas_call(
        _argmin_body,
        grid=grid,
        in_specs=[
            pl.BlockSpec((ROW_TILE, d), lambda i: (i, 0)),
            pl.BlockSpec((k, d), lambda i: (0, 0)),
        ],
        out_specs=[
            pl.BlockSpec((1, 1, ROW_TILE), lambda i: (i, 0, 0)),
            pl.BlockSpec((1, 1), lambda i: (0, 0)),
        ],
        out_shape=[idx_out, acc_out],
        scratch_shapes=[pltpu.VMEM((1, k), jnp.float32),
                        pltpu.VMEM((k, d), jnp.bfloat16)],
    )(flat, embed)


def _make_gather(k, d, b):
    info = plsc.get_sparse_core_info()
    nw = info.num_cores * info.num_subcores
    ch = 128
    b_per_w = b // nw
    n_chunks = b_per_w // ch
    mesh = plsc.VectorSubcoreMesh(core_axis_name="c", subcore_axis_name="s")

    @functools.partial(
        pl.kernel,
        mesh=mesh,
        out_type=jax.ShapeDtypeStruct((b, d), jnp.float32),
        scratch_types=[
            pltpu.VMEM((ch,), jnp.int32),
            pltpu.VMEM((ch, d), jnp.float32),
            pltpu.SemaphoreType.DMA,
        ],
    )
    def gather_k(table_hbm, idx_hbm, out_hbm, idx_v, rows_v, sem):
        wid = lax.axis_index("s") * info.num_cores + lax.axis_index("c")
        base = wid * b_per_w

        def chunk(c, carry):
            off = base + c * ch
            pltpu.sync_copy(idx_hbm.at[pl.ds(off, ch)], idx_v)
            pltpu.async_copy(table_hbm.at[idx_v], rows_v, sem).wait()
            pltpu.sync_copy(rows_v, out_hbm.at[pl.ds(off, ch)])
            return carry

        lax.fori_loop(0, n_chunks, chunk, 0)

    return gather_k


def kernel(z, embed):
    b, n, d = z.shape
    k = embed.shape[0]
    m = b * n
    flat = z.reshape(m, d)
    idx3, acc = _argmin_call(flat, embed)
    indices = idx3.reshape(m)
    zq_flat = _make_gather(k, d, m)(embed, indices)
    loss = acc[0, 0] * (2.0 / (m * d))
    return zq_flat.reshape(b, n, d), indices.reshape(b, n), loss
